# baseline (speedup 1.0000x reference)
import functools

import jax
import jax.numpy as jnp
from jax import lax
from jax.experimental import pallas as pl
from jax.experimental.pallas import tpu as pltpu
from jax.experimental.pallas import tpu_sc as plsc

EMBED_DIM = 64
NUM_CORES = 2
NUM_SUBCORES = 16
NUM_WORKERS = NUM_CORES * NUM_SUBCORES

@functools.lru_cache(maxsize=None)
def _make_gather(batch: int, hist: int, CHUNK: int):
    total_rows = batch * hist
    assert total_rows % (NUM_WORKERS * 2 * CHUNK) == 0
    rows_per_worker = total_rows // NUM_WORKERS
    num_chunks = rows_per_worker // CHUNK
    num_pairs = num_chunks // 2
    mesh = plsc.VectorSubcoreMesh(core_axis_name="c", subcore_axis_name="s")

    @functools.partial(
        pl.kernel,
        mesh=mesh,
        out_type=jax.ShapeDtypeStruct((total_rows, EMBED_DIM), jnp.float32),
        compiler_params=pltpu.CompilerParams(use_tc_tiling_on_sc=False),
        scratch_types=[
            pltpu.VMEM((CHUNK,), jnp.int32),
            pltpu.VMEM((CHUNK,), jnp.int32),
            pltpu.VMEM((CHUNK, EMBED_DIM), jnp.float32),
            pltpu.VMEM((CHUNK, EMBED_DIM), jnp.float32),
            pltpu.SemaphoreType.DMA,
            pltpu.SemaphoreType.DMA,
            pltpu.SemaphoreType.DMA,
            pltpu.SemaphoreType.DMA,
            pltpu.SemaphoreType.DMA,
            pltpu.SemaphoreType.DMA,
        ],
    )
    def gather_kernel(idx_hbm, table_hbm, out_hbm,
                      idx0, idx1, rows0, rows1,
                      isem0, isem1, gsem0, gsem1, osem0, osem1):
        wid = lax.axis_index("s") * NUM_CORES + lax.axis_index("c")
        base = wid * rows_per_worker

        pltpu.async_copy(idx_hbm.at[pl.ds(base, CHUNK)], idx0, isem0)
        pltpu.async_copy(idx_hbm.at[pl.ds(base + CHUNK, CHUNK)], idx1, isem1)

        bufs = ((idx0, rows0, isem0, gsem0, osem0),
                (idx1, rows1, isem1, gsem1, osem1))

        def body(i, _):
            for b, (idx_v, rows_v, isem, gsem, osem) in enumerate(bufs):
                c = 2 * i + b
                off = base + c * CHUNK
                pltpu.make_async_copy(idx_hbm.at[pl.ds(off, CHUNK)], idx_v,
                                      isem).wait()
                @pl.when(i > 0)
                def _():
                    pltpu.make_async_copy(
                        rows_v, out_hbm.at[pl.ds(off - 2 * CHUNK, CHUNK)],
                        osem).wait()
                pltpu.async_copy(table_hbm.at[idx_v], rows_v, gsem).wait()
                @pl.when(c + 2 < num_chunks)
                def _():
                    pltpu.async_copy(
                        idx_hbm.at[pl.ds(off + 2 * CHUNK, CHUNK)], idx_v, isem)
                pltpu.async_copy(rows_v, out_hbm.at[pl.ds(off, CHUNK)], osem)
            return 0

        lax.fori_loop(0, num_pairs, body, 0)

        tail = base + (num_chunks - 2) * CHUNK
        pltpu.make_async_copy(rows0, out_hbm.at[pl.ds(tail, CHUNK)],
                              osem0).wait()
        pltpu.make_async_copy(rows1, out_hbm.at[pl.ds(tail + CHUNK, CHUNK)],
                              osem1).wait()

    return gather_kernel


def kernel(x, table):
    batch, hist = x.shape
    nsplit = 5 if hist % 5 == 0 else 1
    hs = hist // nsplit
    chunk = 640 if (batch * hs) % (NUM_WORKERS * 2 * 640) == 0 else 800
    gather = _make_gather(batch, hs, chunk)
    outs = []
    for k in range(nsplit):
        idx = x[:, k * hs:(k + 1) * hs].reshape(batch * hs).astype(jnp.int32)
        out = gather(idx, table)
        outs.append(out.reshape(batch, hs, EMBED_DIM))
    if nsplit == 1:
        return outs[0]
    return jnp.concatenate(outs, axis=1)


# speedup vs baseline: 1.1343x; 1.1343x over previous
import functools

import jax
import jax.numpy as jnp
from jax import lax
from jax.experimental import pallas as pl
from jax.experimental.pallas import tpu as pltpu
from jax.experimental.pallas import tpu_sc as plsc

EMBED_DIM = 64
NUM_CORES = 2
NUM_SUBCORES = 16
NUM_WORKERS = NUM_CORES * NUM_SUBCORES

@functools.lru_cache(maxsize=None)
def _make_gather(batch: int, hist: int, CHUNK: int):
    total_rows = batch * hist
    assert total_rows % (NUM_WORKERS * 2 * CHUNK) == 0
    rows_per_worker = total_rows // NUM_WORKERS
    num_chunks = rows_per_worker // CHUNK
    num_pairs = num_chunks // 2
    mesh = plsc.VectorSubcoreMesh(core_axis_name="c", subcore_axis_name="s")

    @functools.partial(
        pl.kernel,
        mesh=mesh,
        out_type=jax.ShapeDtypeStruct((total_rows, EMBED_DIM), jnp.float32),
        compiler_params=pltpu.CompilerParams(use_tc_tiling_on_sc=False),
        scratch_types=[
            pltpu.VMEM((CHUNK,), jnp.int32),
            pltpu.VMEM((CHUNK,), jnp.int32),
            pltpu.VMEM((CHUNK, EMBED_DIM), jnp.float32),
            pltpu.VMEM((CHUNK, EMBED_DIM), jnp.float32),
            pltpu.SemaphoreType.DMA,
            pltpu.SemaphoreType.DMA,
            pltpu.SemaphoreType.DMA,
            pltpu.SemaphoreType.DMA,
            pltpu.SemaphoreType.DMA,
            pltpu.SemaphoreType.DMA,
        ],
    )
    def gather_kernel(idx_hbm, table_hbm, out_hbm,
                      idx0, idx1, rows0, rows1,
                      isem0, isem1, gsem0, gsem1, osem0, osem1):
        wid = lax.axis_index("s") * NUM_CORES + lax.axis_index("c")
        base = wid * rows_per_worker

        pltpu.async_copy(idx_hbm.at[pl.ds(base, CHUNK)], idx0, isem0)
        pltpu.async_copy(idx_hbm.at[pl.ds(base + CHUNK, CHUNK)], idx1, isem1)

        bufs = ((idx0, rows0, isem0, gsem0, osem0),
                (idx1, rows1, isem1, gsem1, osem1))

        def body(i, _):
            for b, (idx_v, rows_v, isem, gsem, osem) in enumerate(bufs):
                c = 2 * i + b
                off = base + c * CHUNK
                pltpu.make_async_copy(idx_hbm.at[pl.ds(off, CHUNK)], idx_v,
                                      isem).wait()
                @pl.when(i > 0)
                def _():
                    pltpu.make_async_copy(
                        rows_v, out_hbm.at[pl.ds(off - 2 * CHUNK, CHUNK)],
                        osem).wait()
                pltpu.async_copy(table_hbm.at[idx_v], rows_v, gsem).wait()
                @pl.when(c + 2 < num_chunks)
                def _():
                    pltpu.async_copy(
                        idx_hbm.at[pl.ds(off + 2 * CHUNK, CHUNK)], idx_v, isem)
                pltpu.async_copy(rows_v, out_hbm.at[pl.ds(off, CHUNK)], osem)
            return 0

        lax.fori_loop(0, num_pairs, body, 0)

        tail = base + (num_chunks - 2) * CHUNK
        pltpu.make_async_copy(rows0, out_hbm.at[pl.ds(tail, CHUNK)],
                              osem0).wait()
        pltpu.make_async_copy(rows1, out_hbm.at[pl.ds(tail + CHUNK, CHUNK)],
                              osem1).wait()

    return gather_kernel


def kernel(x, table):
    batch, hist = x.shape
    idx = x.reshape(batch * hist).astype(jnp.int32)
    out = _make_gather(batch, hist, 800)(idx, table)
    return out.reshape(batch, hist, EMBED_DIM)


# speedup vs baseline: 1.1349x; 1.0006x over previous
import functools

import jax
import jax.numpy as jnp
from jax import lax
from jax.experimental import pallas as pl
from jax.experimental.pallas import tpu as pltpu
from jax.experimental.pallas import tpu_sc as plsc

EMBED_DIM = 64
NUM_CORES = 2
NUM_SUBCORES = 16
NUM_WORKERS = NUM_CORES * NUM_SUBCORES

@functools.lru_cache(maxsize=None)
def _make_gather(batch: int, hist: int, CHUNK: int):
    total_rows = batch * hist
    assert total_rows % (NUM_WORKERS * 2 * CHUNK) == 0
    rows_per_worker = total_rows // NUM_WORKERS
    num_chunks = rows_per_worker // CHUNK
    num_pairs = num_chunks // 2
    mesh = plsc.VectorSubcoreMesh(core_axis_name="c", subcore_axis_name="s")

    @functools.partial(
        pl.kernel,
        mesh=mesh,
        out_type=jax.ShapeDtypeStruct((total_rows, EMBED_DIM), jnp.float32),
        compiler_params=pltpu.CompilerParams(use_tc_tiling_on_sc=False),
        scratch_types=[
            pltpu.VMEM((CHUNK,), jnp.int32),
            pltpu.VMEM((CHUNK,), jnp.int32),
            pltpu.VMEM((CHUNK, EMBED_DIM), jnp.float32),
            pltpu.VMEM((CHUNK, EMBED_DIM), jnp.float32),
            pltpu.SemaphoreType.DMA,
            pltpu.SemaphoreType.DMA,
            pltpu.SemaphoreType.DMA,
            pltpu.SemaphoreType.DMA,
            pltpu.SemaphoreType.DMA,
            pltpu.SemaphoreType.DMA,
        ],
    )
    def gather_kernel(idx_hbm, table_hbm, out_hbm,
                      idx0, idx1, rows0, rows1,
                      isem0, isem1, gsem0, gsem1, osem0, osem1):
        wid = lax.axis_index("s") * NUM_CORES + lax.axis_index("c")
        base = wid * rows_per_worker

        pltpu.async_copy(idx_hbm.at[pl.ds(base, CHUNK)], idx0, isem0)
        pltpu.async_copy(idx_hbm.at[pl.ds(base + CHUNK, CHUNK)], idx1, isem1)

        bufs = ((idx0, rows0, isem0, gsem0, osem0),
                (idx1, rows1, isem1, gsem1, osem1))

        def body(i, _):
            for b, (idx_v, rows_v, isem, gsem, osem) in enumerate(bufs):
                c = 2 * i + b
                off = base + c * CHUNK
                pltpu.make_async_copy(idx_hbm.at[pl.ds(off, CHUNK)], idx_v,
                                      isem).wait()
                @pl.when(i > 0)
                def _():
                    pltpu.make_async_copy(
                        rows_v, out_hbm.at[pl.ds(off - 2 * CHUNK, CHUNK)],
                        osem).wait()
                pltpu.async_copy(table_hbm.at[idx_v], rows_v, gsem).wait()
                @pl.when(c + 2 < num_chunks)
                def _():
                    pltpu.async_copy(
                        idx_hbm.at[pl.ds(off + 2 * CHUNK, CHUNK)], idx_v, isem)
                pltpu.async_copy(rows_v, out_hbm.at[pl.ds(off, CHUNK)], osem)
            return 0

        lax.fori_loop(0, num_pairs, body, 0)

        tail = base + (num_chunks - 2) * CHUNK
        pltpu.make_async_copy(rows0, out_hbm.at[pl.ds(tail, CHUNK)],
                              osem0).wait()
        pltpu.make_async_copy(rows1, out_hbm.at[pl.ds(tail + CHUNK, CHUNK)],
                              osem1).wait()

    return gather_kernel


def kernel(x, table):
    batch, hist = x.shape
    idx = x.reshape(batch * hist).astype(jnp.int32)
    out = _make_gather(batch, hist, 800)(idx, table)
    out = jax.lax.optimization_barrier(
        out.reshape(batch * hist // 2, 2 * EMBED_DIM))
    return out.reshape(batch, hist, EMBED_DIM)
